# Initial kernel scaffold; baseline (speedup 1.0000x reference)
import jax
import jax.numpy as jnp
from jax.experimental import pallas as pl


def kernel(elems, atomic_energy, scales, shifts, elem_lookup):
    raise NotImplementedError("write your pallas kernel here")


# speedup vs baseline: 588.9864x; 588.9864x over previous
import functools

import jax
import jax.numpy as jnp
from jax import lax
from jax.experimental import pallas as pl
from jax.experimental.pallas import tpu as pltpu
from jax.experimental.pallas import tpu_sc as plsc

_LANES = 16
_TABLE_PAD = 128


def _scale_shift_sc(elems, energy, lut, tsc, tsh):
    n = elems.shape[0]
    mesh = plsc.VectorSubcoreMesh(core_axis_name="c", subcore_axis_name="s")
    n_workers = mesh.num_cores * mesh.num_subcores
    per_w = n // n_workers
    assert n % n_workers == 0 and per_w % _LANES == 0

    def body(elems_hbm, energy_hbm, lut_hbm, tsc_hbm, tsh_hbm, out_hbm,
             ev, av, ov, lut_v, tsc_v, tsh_v, fsc_v, fsh_v):
        wid = lax.axis_index("s") * mesh.num_cores + lax.axis_index("c")
        base = wid * per_w

        pltpu.sync_copy(lut_hbm, lut_v)
        pltpu.sync_copy(tsc_hbm, tsc_v)
        pltpu.sync_copy(tsh_hbm, tsh_v)
        pltpu.sync_copy(elems_hbm.at[pl.ds(base, per_w)], ev)
        pltpu.sync_copy(energy_hbm.at[pl.ds(base, per_w)], av)

        @pl.loop(0, _TABLE_PAD // _LANES)
        def _fuse(j):
            o = j * _LANES
            d = lut_v[pl.ds(o, _LANES)]
            fsc_v[pl.ds(o, _LANES)] = plsc.load_gather(tsc_v, [d])
            fsh_v[pl.ds(o, _LANES)] = plsc.load_gather(tsh_v, [d])

        @pl.loop(0, per_w // _LANES, unroll=8)
        def _main(j):
            o = j * _LANES
            idx = ev[pl.ds(o, _LANES)]
            sc = plsc.load_gather(fsc_v, [idx])
            sh = plsc.load_gather(fsh_v, [idx])
            ov[pl.ds(o, _LANES)] = av[pl.ds(o, _LANES)] * sc + sh

        pltpu.sync_copy(ov, out_hbm.at[pl.ds(base, per_w)])

    run = pl.kernel(
        body,
        out_type=jax.ShapeDtypeStruct((n,), jnp.float32),
        mesh=mesh,
        compiler_params=pltpu.CompilerParams(needs_layout_passes=False),
        scratch_types=[
            pltpu.VMEM((per_w,), jnp.int32),
            pltpu.VMEM((per_w,), jnp.float32),
            pltpu.VMEM((per_w,), jnp.float32),
            pltpu.VMEM((_TABLE_PAD,), jnp.int32),
            pltpu.VMEM((_TABLE_PAD,), jnp.float32),
            pltpu.VMEM((_TABLE_PAD,), jnp.float32),
            pltpu.VMEM((_TABLE_PAD,), jnp.float32),
            pltpu.VMEM((_TABLE_PAD,), jnp.float32),
        ],
    )
    return run(elems, energy, lut, tsc, tsh)


def kernel(elems, atomic_energy, scales, shifts, elem_lookup):
    t = elem_lookup.shape[0]
    lut = jnp.pad(elem_lookup.astype(jnp.int32), (0, _TABLE_PAD - t))
    tsc = jnp.pad(scales, (0, _TABLE_PAD - scales.shape[0]))
    tsh = jnp.pad(shifts, (0, _TABLE_PAD - shifts.shape[0]))
    return _scale_shift_sc(elems, atomic_energy, lut, tsc, tsh)


# speedup vs baseline: 893.3752x; 1.5168x over previous
import functools

import jax
import jax.numpy as jnp
from jax import lax
from jax.experimental import pallas as pl
from jax.experimental.pallas import tpu as pltpu
from jax.experimental.pallas import tpu_sc as plsc

_LANES = 16
_TABLE_PAD = 128


def _scale_shift_sc(elems, energy, lut, tsc, tsh):
    n = elems.shape[0]
    mesh = plsc.VectorSubcoreMesh(core_axis_name="c", subcore_axis_name="s")
    n_workers = mesh.num_cores * mesh.num_subcores
    per_w = n // n_workers
    assert n % n_workers == 0 and per_w % _LANES == 0

    def body(elems_hbm, energy_hbm, lut_hbm, tsc_hbm, tsh_hbm, out_hbm,
             ev, av, ov, lut_v, tsc_v, tsh_v, fsc_v, fsh_v):
        wid = lax.axis_index("s") * mesh.num_cores + lax.axis_index("c")
        base = wid * per_w

        pltpu.sync_copy(lut_hbm, lut_v)
        pltpu.sync_copy(tsc_hbm, tsc_v)
        pltpu.sync_copy(tsh_hbm, tsh_v)
        pltpu.sync_copy(elems_hbm.at[pl.ds(base, per_w)], ev)
        pltpu.sync_copy(energy_hbm.at[pl.ds(base, per_w)], av)

        @pl.loop(0, _TABLE_PAD // _LANES)
        def _fuse(j):
            o = j * _LANES
            d = lut_v[pl.ds(o, _LANES)]
            fsc_v[pl.ds(o, _LANES)] = plsc.load_gather(tsc_v, [d])
            fsh_v[pl.ds(o, _LANES)] = plsc.load_gather(tsh_v, [d])

        @plsc.parallel_loop(0, per_w // _LANES, unroll=8)
        def _main(j):
            o = j * _LANES
            idx = ev[pl.ds(o, _LANES)]
            sc = plsc.load_gather(fsc_v, [idx])
            sh = plsc.load_gather(fsh_v, [idx])
            ov[pl.ds(o, _LANES)] = av[pl.ds(o, _LANES)] * sc + sh

        pltpu.sync_copy(ov, out_hbm.at[pl.ds(base, per_w)])

    run = pl.kernel(
        body,
        out_type=jax.ShapeDtypeStruct((n,), jnp.float32),
        mesh=mesh,
        compiler_params=pltpu.CompilerParams(needs_layout_passes=False),
        scratch_types=[
            pltpu.VMEM((per_w,), jnp.int32),
            pltpu.VMEM((per_w,), jnp.float32),
            pltpu.VMEM((per_w,), jnp.float32),
            pltpu.VMEM((_TABLE_PAD,), jnp.int32),
            pltpu.VMEM((_TABLE_PAD,), jnp.float32),
            pltpu.VMEM((_TABLE_PAD,), jnp.float32),
            pltpu.VMEM((_TABLE_PAD,), jnp.float32),
            pltpu.VMEM((_TABLE_PAD,), jnp.float32),
        ],
    )
    return run(elems, energy, lut, tsc, tsh)


def kernel(elems, atomic_energy, scales, shifts, elem_lookup):
    t = elem_lookup.shape[0]
    lut = jnp.pad(elem_lookup.astype(jnp.int32), (0, _TABLE_PAD - t))
    tsc = jnp.pad(scales, (0, _TABLE_PAD - scales.shape[0]))
    tsh = jnp.pad(shifts, (0, _TABLE_PAD - shifts.shape[0]))
    return _scale_shift_sc(elems, atomic_energy, lut, tsc, tsh)


# speedup vs baseline: 1019.8156x; 1.1415x over previous
import functools

import jax
import jax.numpy as jnp
from jax import lax
from jax.experimental import pallas as pl
from jax.experimental.pallas import tpu as pltpu
from jax.experimental.pallas import tpu_sc as plsc

_LANES = 16
_TABLE_PAD = 128


_NCHUNK = 4


def _scale_shift_sc(elems, energy, lut, tsc, tsh):
    n = elems.shape[0]
    mesh = plsc.VectorSubcoreMesh(core_axis_name="c", subcore_axis_name="s")
    n_workers = mesh.num_cores * mesh.num_subcores
    per_w = n // n_workers
    ch = per_w // _NCHUNK
    assert n % n_workers == 0 and per_w % (_NCHUNK * _LANES) == 0

    def body(elems_hbm, energy_hbm, lut_hbm, tsc_hbm, tsh_hbm, out_hbm,
             ev0, ev1, av0, av1, ov0, ov1,
             lut_v, tsc_v, tsh_v, fsc_v, fsh_v,
             se0, se1, sa0, sa1, so0, so1):
        wid = lax.axis_index("s") * mesh.num_cores + lax.axis_index("c")
        base = wid * per_w
        evs, avs, ovs = [ev0, ev1], [av0, av1], [ov0, ov1]
        ses, sas, sos = [se0, se1], [sa0, sa1], [so0, so1]

        def fetch(c):
            b = c % 2
            src = pl.ds(base + c * ch, ch)
            return (pltpu.async_copy(elems_hbm.at[src], evs[b], ses[b]),
                    pltpu.async_copy(energy_hbm.at[src], avs[b], sas[b]))

        in_flight = {0: fetch(0)}

        pltpu.sync_copy(lut_hbm, lut_v)
        pltpu.sync_copy(tsc_hbm, tsc_v)
        pltpu.sync_copy(tsh_hbm, tsh_v)

        @pl.loop(0, _TABLE_PAD // _LANES)
        def _fuse(j):
            o = j * _LANES
            d = lut_v[pl.ds(o, _LANES)]
            fsc_v[pl.ds(o, _LANES)] = plsc.load_gather(tsc_v, [d])
            fsh_v[pl.ds(o, _LANES)] = plsc.load_gather(tsh_v, [d])

        out_flight = {}
        for c in range(_NCHUNK):
            b = c % 2
            if c + 1 < _NCHUNK:
                in_flight[c + 1] = fetch(c + 1)
            for h in in_flight.pop(c):
                h.wait()
            if c - 2 in out_flight:
                out_flight.pop(c - 2).wait()
            ev, av, ov = evs[b], avs[b], ovs[b]

            @plsc.parallel_loop(0, ch // _LANES, unroll=8)
            def _main(j):
                o = j * _LANES
                idx = ev[pl.ds(o, _LANES)]
                sc = plsc.load_gather(fsc_v, [idx])
                sh = plsc.load_gather(fsh_v, [idx])
                ov[pl.ds(o, _LANES)] = av[pl.ds(o, _LANES)] * sc + sh

            out_flight[c] = pltpu.async_copy(
                ov, out_hbm.at[pl.ds(base + c * ch, ch)], sos[b])
        for h in out_flight.values():
            h.wait()

    run = pl.kernel(
        body,
        out_type=jax.ShapeDtypeStruct((n,), jnp.float32),
        mesh=mesh,
        compiler_params=pltpu.CompilerParams(needs_layout_passes=False),
        scratch_types=[
            pltpu.VMEM((ch,), jnp.int32),
            pltpu.VMEM((ch,), jnp.int32),
            pltpu.VMEM((ch,), jnp.float32),
            pltpu.VMEM((ch,), jnp.float32),
            pltpu.VMEM((ch,), jnp.float32),
            pltpu.VMEM((ch,), jnp.float32),
            pltpu.VMEM((_TABLE_PAD,), jnp.int32),
            pltpu.VMEM((_TABLE_PAD,), jnp.float32),
            pltpu.VMEM((_TABLE_PAD,), jnp.float32),
            pltpu.VMEM((_TABLE_PAD,), jnp.float32),
            pltpu.VMEM((_TABLE_PAD,), jnp.float32),
            pltpu.SemaphoreType.DMA,
            pltpu.SemaphoreType.DMA,
            pltpu.SemaphoreType.DMA,
            pltpu.SemaphoreType.DMA,
            pltpu.SemaphoreType.DMA,
            pltpu.SemaphoreType.DMA,
        ],
    )
    return run(elems, energy, lut, tsc, tsh)


def kernel(elems, atomic_energy, scales, shifts, elem_lookup):
    t = elem_lookup.shape[0]
    lut = jnp.pad(elem_lookup.astype(jnp.int32), (0, _TABLE_PAD - t))
    tsc = jnp.pad(scales, (0, _TABLE_PAD - scales.shape[0]))
    tsh = jnp.pad(shifts, (0, _TABLE_PAD - shifts.shape[0]))
    return _scale_shift_sc(elems, atomic_energy, lut, tsc, tsh)


# speedup vs baseline: 1065.8094x; 1.0451x over previous
import functools

import jax
import jax.numpy as jnp
from jax import lax
from jax.experimental import pallas as pl
from jax.experimental.pallas import tpu as pltpu
from jax.experimental.pallas import tpu_sc as plsc

_LANES = 16
_TABLE_PAD = 128


_NCHUNK = 4


def _scale_shift_sc(elems, energy, lut, tsc, tsh):
    n = elems.shape[0]
    mesh = plsc.VectorSubcoreMesh(core_axis_name="c", subcore_axis_name="s")
    n_workers = mesh.num_cores * mesh.num_subcores
    per_w = n // n_workers
    ch = per_w // _NCHUNK
    assert n % n_workers == 0 and per_w % (_NCHUNK * _LANES) == 0

    def body(elems_hbm, energy_hbm, lut_hbm, tsc_hbm, tsh_hbm, out_hbm,
             ev0, ev1, av0, av1, ov0, ov1,
             lut_v, tsc_v, tsh_v, fsc_v, fsh_v,
             se0, se1, sa0, sa1, so0, so1):
        wid = lax.axis_index("s") * mesh.num_cores + lax.axis_index("c")
        base = wid * per_w
        evs, avs, ovs = [ev0, ev1], [av0, av1], [ov0, ov1]
        ses, sas, sos = [se0, se1], [sa0, sa1], [so0, so1]

        def fetch(c):
            b = c % 2
            src = pl.ds(base + c * ch, ch)
            return (pltpu.async_copy(elems_hbm.at[src], evs[b], ses[b]),
                    pltpu.async_copy(energy_hbm.at[src], avs[b], sas[b]))

        in_flight = {0: fetch(0)}

        t = lut_hbm.shape[0]
        pltpu.sync_copy(lut_hbm, lut_v.at[pl.ds(0, t)])
        pltpu.sync_copy(tsc_hbm, tsc_v.at[pl.ds(0, t)])
        pltpu.sync_copy(tsh_hbm, tsh_v.at[pl.ds(0, t)])

        @pl.loop(0, _TABLE_PAD // _LANES)
        def _fuse(j):
            o = j * _LANES
            d = lut_v[pl.ds(o, _LANES)]
            d = jnp.minimum(jnp.maximum(d, 0), t - 1)
            fsc_v[pl.ds(o, _LANES)] = plsc.load_gather(tsc_v, [d])
            fsh_v[pl.ds(o, _LANES)] = plsc.load_gather(tsh_v, [d])

        out_flight = {}
        for c in range(_NCHUNK):
            b = c % 2
            if c + 1 < _NCHUNK:
                in_flight[c + 1] = fetch(c + 1)
            for h in in_flight.pop(c):
                h.wait()
            if c - 2 in out_flight:
                out_flight.pop(c - 2).wait()
            ev, av, ov = evs[b], avs[b], ovs[b]

            @plsc.parallel_loop(0, ch // _LANES, unroll=8)
            def _main(j):
                o = j * _LANES
                idx = ev[pl.ds(o, _LANES)]
                sc = plsc.load_gather(fsc_v, [idx])
                sh = plsc.load_gather(fsh_v, [idx])
                ov[pl.ds(o, _LANES)] = av[pl.ds(o, _LANES)] * sc + sh

            out_flight[c] = pltpu.async_copy(
                ov, out_hbm.at[pl.ds(base + c * ch, ch)], sos[b])
        for h in out_flight.values():
            h.wait()

    run = pl.kernel(
        body,
        out_type=jax.ShapeDtypeStruct((n,), jnp.float32),
        mesh=mesh,
        compiler_params=pltpu.CompilerParams(needs_layout_passes=False),
        scratch_types=[
            pltpu.VMEM((ch,), jnp.int32),
            pltpu.VMEM((ch,), jnp.int32),
            pltpu.VMEM((ch,), jnp.float32),
            pltpu.VMEM((ch,), jnp.float32),
            pltpu.VMEM((ch,), jnp.float32),
            pltpu.VMEM((ch,), jnp.float32),
            pltpu.VMEM((_TABLE_PAD,), jnp.int32),
            pltpu.VMEM((_TABLE_PAD,), jnp.float32),
            pltpu.VMEM((_TABLE_PAD,), jnp.float32),
            pltpu.VMEM((_TABLE_PAD,), jnp.float32),
            pltpu.VMEM((_TABLE_PAD,), jnp.float32),
            pltpu.SemaphoreType.DMA,
            pltpu.SemaphoreType.DMA,
            pltpu.SemaphoreType.DMA,
            pltpu.SemaphoreType.DMA,
            pltpu.SemaphoreType.DMA,
            pltpu.SemaphoreType.DMA,
        ],
    )
    return run(elems, energy, lut, tsc, tsh)


def kernel(elems, atomic_energy, scales, shifts, elem_lookup):
    return _scale_shift_sc(elems, atomic_energy,
                           elem_lookup.astype(jnp.int32), scales, shifts)
